# Initial kernel scaffold; baseline (speedup 1.0000x reference)
import jax
import jax.numpy as jnp
from jax.experimental import pallas as pl


def kernel(input_encoded, timesteps, pos_encoding):
    raise NotImplementedError("write your pallas kernel here")


# speedup vs baseline: 1.7594x; 1.7594x over previous
import functools

import jax
import jax.numpy as jnp
from jax import lax
from jax.experimental import pallas as pl
from jax.experimental.pallas import tpu as pltpu
from jax.experimental.pallas import tpu_sc as plsc

B, S, T, D, L = 64, 50, 200, 64, 5000
ROWS_PER_BATCH = S * T
N_ROWS = B * ROWS_PER_BATCH
NUM_WORKERS = 32
BATCHES_PER_WORKER = B // NUM_WORKERS
CHUNK = 80
NCHUNKS = ROWS_PER_BATCH // CHUNK
LANES = 16


def _sc_body(in_hbm, ts_hbm, table_hbm, out_hbm,
             ts_buf, idx_buf, in_buf, pe_buf, sem):
    wid = lax.axis_index("s") * 2 + lax.axis_index("c")

    for bb in range(BATCHES_PER_WORKER):
        b = wid * BATCHES_PER_WORKER + bb
        row0 = b * ROWS_PER_BATCH

        pltpu.sync_copy(ts_hbm.at[pl.ds(row0, ROWS_PER_BATCH)], ts_buf)

        def min_step(j, m):
            return jnp.minimum(m, ts_buf[pl.ds(j * LANES, LANES)])

        m0 = jnp.full((LANES,), jnp.iinfo(jnp.int32).max, dtype=jnp.int32)
        m = lax.fori_loop(0, ROWS_PER_BATCH // LANES, min_step, m0)
        iota = lax.broadcasted_iota(jnp.int32, (LANES,), 0)
        for k in (8, 4, 2, 1):
            perm = jnp.take_along_axis(m, iota ^ k, axis=0,
                                       mode="promise_in_bounds")
            m = jnp.minimum(m, perm)
        min_splat = m

        def chunk_step(c, carry):
            base = row0 + c * CHUNK
            for j in range(CHUNK // LANES):
                idx_buf[pl.ds(j * LANES, LANES)] = (
                    ts_buf[pl.ds(c * CHUNK + j * LANES, LANES)] - min_splat)
            pltpu.sync_copy(in_hbm.at[pl.ds(base, CHUNK)], in_buf)
            pltpu.async_copy(table_hbm.at[idx_buf], pe_buf, sem).wait()

            def add_row(r, carry2):
                for l in range(D // LANES):
                    sl = pl.ds(l * LANES, LANES)
                    in_buf[r, sl] = in_buf[r, sl] + pe_buf[r, sl]
                return carry2

            lax.fori_loop(0, CHUNK, add_row, 0)
            pltpu.sync_copy(in_buf, out_hbm.at[pl.ds(base, CHUNK)])
            return carry

        lax.fori_loop(0, NCHUNKS, chunk_step, 0)


@functools.partial(jax.jit, static_argnames=())
def kernel(input_encoded, timesteps, pos_encoding):
    in2d = input_encoded.reshape(N_ROWS, D)
    ts1d = timesteps.reshape(N_ROWS)

    mesh = plsc.VectorSubcoreMesh(core_axis_name="c", subcore_axis_name="s")
    run = pl.kernel(
        _sc_body,
        out_type=jax.ShapeDtypeStruct((N_ROWS, D), jnp.float32),
        mesh=mesh,
        scratch_types=[
            pltpu.VMEM((ROWS_PER_BATCH,), jnp.int32),
            pltpu.VMEM((CHUNK,), jnp.int32),
            pltpu.VMEM((CHUNK, D), jnp.float32),
            pltpu.VMEM((CHUNK, D), jnp.float32),
            pltpu.SemaphoreType.DMA,
        ],
        compiler_params=pltpu.CompilerParams(use_tc_tiling_on_sc=False),
    )
    out2d = run(in2d, ts1d, pos_encoding)
    return out2d.reshape(B, S, T, D)


# speedup vs baseline: 1.8601x; 1.0573x over previous
import functools

import jax
import jax.numpy as jnp
from jax import lax
from jax.experimental import pallas as pl
from jax.experimental.pallas import tpu as pltpu
from jax.experimental.pallas import tpu_sc as plsc

B, S, T, D, L = 64, 50, 200, 64, 5000
ROWS_PER_BATCH = S * T
N_ROWS = B * ROWS_PER_BATCH
NUM_WORKERS = 32
BATCHES_PER_WORKER = B // NUM_WORKERS
CHUNK = 80
NCHUNKS = ROWS_PER_BATCH // CHUNK
LANES = 16


def _sc_body(in_hbm, ts_hbm, table_hbm, out_hbm,
             ts_buf, idx_buf, in_buf, pe_buf, sem):
    wid = lax.axis_index("s") * 2 + lax.axis_index("c")

    for bb in range(BATCHES_PER_WORKER):
        b = wid * BATCHES_PER_WORKER + bb
        row0 = b * ROWS_PER_BATCH

        pltpu.sync_copy(ts_hbm.at[pl.ds(row0, ROWS_PER_BATCH)], ts_buf)

        def min_step(j, m):
            return jnp.minimum(m, ts_buf[pl.ds(j * LANES, LANES)])

        m0 = jnp.full((LANES,), jnp.iinfo(jnp.int32).max, dtype=jnp.int32)
        m = lax.fori_loop(0, ROWS_PER_BATCH // LANES, min_step, m0)
        iota = lax.broadcasted_iota(jnp.int32, (LANES,), 0)
        for k in (8, 4, 2, 1):
            perm = jnp.take_along_axis(m, iota ^ k, axis=0,
                                       mode="promise_in_bounds")
            m = jnp.minimum(m, perm)
        min_splat = m

        def chunk_step(c, carry):
            base = row0 + c * CHUNK
            for j in range(CHUNK // LANES):
                idx_buf[pl.ds(j * LANES, LANES)] = (
                    ts_buf[pl.ds(c * CHUNK + j * LANES, LANES)] - min_splat)
            pltpu.sync_copy(in_hbm.at[pl.ds(base, CHUNK)], in_buf)
            pltpu.async_copy(table_hbm.at[idx_buf], in_buf, sem,
                             add=True).wait()
            pltpu.sync_copy(in_buf, out_hbm.at[pl.ds(base, CHUNK)])
            return carry

        lax.fori_loop(0, NCHUNKS, chunk_step, 0)


@functools.partial(jax.jit, static_argnames=())
def kernel(input_encoded, timesteps, pos_encoding):
    in2d = input_encoded.reshape(N_ROWS, D)
    ts1d = timesteps.reshape(N_ROWS)

    mesh = plsc.VectorSubcoreMesh(core_axis_name="c", subcore_axis_name="s")
    run = pl.kernel(
        _sc_body,
        out_type=jax.ShapeDtypeStruct((N_ROWS, D), jnp.float32),
        mesh=mesh,
        scratch_types=[
            pltpu.VMEM((ROWS_PER_BATCH,), jnp.int32),
            pltpu.VMEM((CHUNK,), jnp.int32),
            pltpu.VMEM((CHUNK, D), jnp.float32),
            pltpu.VMEM((CHUNK, D), jnp.float32),
            pltpu.SemaphoreType.DMA,
        ],
        compiler_params=pltpu.CompilerParams(use_tc_tiling_on_sc=False),
    )
    out2d = run(in2d, ts1d, pos_encoding)
    return out2d.reshape(B, S, T, D)


# speedup vs baseline: 2.4558x; 1.3203x over previous
import functools

import jax
import jax.numpy as jnp
from jax import lax
from jax.experimental import pallas as pl
from jax.experimental.pallas import tpu as pltpu
from jax.experimental.pallas import tpu_sc as plsc

B, S, T, D, L = 64, 50, 200, 64, 5000
ROWS_PER_BATCH = S * T
N_ROWS = B * ROWS_PER_BATCH
NUM_WORKERS = 32
BATCHES_PER_WORKER = B // NUM_WORKERS
CHUNK = 400
GCHUNK = 80
NGATHER = CHUNK // GCHUNK
NCHUNKS = ROWS_PER_BATCH // CHUNK
LANES = 16


def _sc_body(in_hbm, ts_hbm, table_hbm, out_hbm,
             ts_buf, idx_bufs, in_bufs, sem_in, sem_out, sem_g):
    wid = lax.axis_index("s") * 2 + lax.axis_index("c")

    for bb in range(BATCHES_PER_WORKER):
        b = wid * BATCHES_PER_WORKER + bb
        row0 = b * ROWS_PER_BATCH

        pltpu.sync_copy(ts_hbm.at[pl.ds(row0, ROWS_PER_BATCH)], ts_buf)

        def min_step(j, m):
            return jnp.minimum(m, ts_buf[pl.ds(j * LANES, LANES)])

        m0 = jnp.full((LANES,), jnp.iinfo(jnp.int32).max, dtype=jnp.int32)
        m = lax.fori_loop(0, ROWS_PER_BATCH // LANES, min_step, m0)
        iota = lax.broadcasted_iota(jnp.int32, (LANES,), 0)
        for k in (8, 4, 2, 1):
            perm = jnp.take_along_axis(m, iota ^ k, axis=0,
                                       mode="promise_in_bounds")
            m = jnp.minimum(m, perm)
        min_splat = m

        def compute_idx(j, p):
            for u in range(CHUNK // LANES):
                idx_bufs[p][pl.ds(u * LANES, LANES)] = (
                    ts_buf[pl.ds(j * CHUNK + u * LANES, LANES)] - min_splat)

        def in_copy(j, p):
            return pltpu.make_async_copy(
                in_hbm.at[pl.ds(row0 + j * CHUNK, CHUNK)],
                in_bufs[p], sem_in[p])

        def out_copy(j, p):
            return pltpu.make_async_copy(
                in_bufs[p], out_hbm.at[pl.ds(row0 + j * CHUNK, CHUNK)],
                sem_out[p])

        def chunk_body(j, p, first):
            in_copy(j, p).wait()
            q = 1 - p

            @pl.when(j + 1 < NCHUNKS)
            def _():
                if not first:
                    out_copy(j, q).wait()
                compute_idx(j + 1, q)
                in_copy(j + 1, q).start()

            descs = [
                pltpu.async_copy(
                    table_hbm.at[idx_bufs[p].at[pl.ds(g * GCHUNK, GCHUNK)]],
                    in_bufs[p].at[pl.ds(g * GCHUNK, GCHUNK)],
                    sem_g, add=True)
                for g in range(NGATHER)
            ]
            for d in descs:
                d.wait()
            out_copy(j, p).start()

        compute_idx(0, 0)
        in_copy(0, 0).start()
        chunk_body(0, 0, first=True)

        def pair(i, carry):
            chunk_body(2 * i - 1, 1, first=False)
            chunk_body(2 * i, 0, first=False)
            return carry

        lax.fori_loop(1, (NCHUNKS + 1) // 2, pair, 0)

        out_copy(NCHUNKS - 2, 1).wait()
        out_copy(NCHUNKS - 1, 0).wait()


@functools.partial(jax.jit, static_argnames=())
def kernel(input_encoded, timesteps, pos_encoding):
    in2d = input_encoded.reshape(N_ROWS, D)
    ts1d = timesteps.reshape(N_ROWS)

    mesh = plsc.VectorSubcoreMesh(core_axis_name="c", subcore_axis_name="s")
    run = pl.kernel(
        _sc_body,
        out_type=jax.ShapeDtypeStruct((N_ROWS, D), jnp.float32),
        mesh=mesh,
        scratch_types=[
            pltpu.VMEM((ROWS_PER_BATCH,), jnp.int32),
            [pltpu.VMEM((CHUNK,), jnp.int32) for _ in range(2)],
            [pltpu.VMEM((CHUNK, D), jnp.float32) for _ in range(2)],
            [pltpu.SemaphoreType.DMA for _ in range(2)],
            [pltpu.SemaphoreType.DMA for _ in range(2)],
            pltpu.SemaphoreType.DMA,
        ],
        compiler_params=pltpu.CompilerParams(use_tc_tiling_on_sc=False),
    )
    out2d = run(in2d, ts1d, pos_encoding)
    return out2d.reshape(B, S, T, D)


# speedup vs baseline: 2.6235x; 1.0683x over previous
import functools

import jax
import jax.numpy as jnp
from jax import lax
from jax.experimental import pallas as pl
from jax.experimental.pallas import tpu as pltpu
from jax.experimental.pallas import tpu_sc as plsc

B, S, T, D, L = 64, 50, 200, 64, 5000
ROWS_PER_BATCH = S * T
N_ROWS = B * ROWS_PER_BATCH
NUM_WORKERS = 32
BATCHES_PER_WORKER = B // NUM_WORKERS
CHUNK = 400
GCHUNK = 80
NGATHER = CHUNK // GCHUNK
NCHUNKS = ROWS_PER_BATCH // CHUNK
LANES = 16


def _sc_body(in_hbm, ts_hbm, table_hbm, out_hbm,
             ts_buf, idx_bufs, in_bufs, table_spm, sem_in, sem_out, sem_g):
    wid = lax.axis_index("s") * 2 + lax.axis_index("c")

    @pl.when(lax.axis_index("s") == 0)
    def _():
        pltpu.sync_copy(table_hbm, table_spm)

    plsc.subcore_barrier()

    for bb in range(BATCHES_PER_WORKER):
        b = wid * BATCHES_PER_WORKER + bb
        row0 = b * ROWS_PER_BATCH

        pltpu.sync_copy(ts_hbm.at[pl.ds(row0, ROWS_PER_BATCH)], ts_buf)

        def min_step(j, m):
            return jnp.minimum(m, ts_buf[pl.ds(j * LANES, LANES)])

        m0 = jnp.full((LANES,), jnp.iinfo(jnp.int32).max, dtype=jnp.int32)
        m = lax.fori_loop(0, ROWS_PER_BATCH // LANES, min_step, m0)
        iota = lax.broadcasted_iota(jnp.int32, (LANES,), 0)
        for k in (8, 4, 2, 1):
            perm = jnp.take_along_axis(m, iota ^ k, axis=0,
                                       mode="promise_in_bounds")
            m = jnp.minimum(m, perm)
        min_splat = m

        def compute_idx(j, p):
            for u in range(CHUNK // LANES):
                idx_bufs[p][pl.ds(u * LANES, LANES)] = (
                    ts_buf[pl.ds(j * CHUNK + u * LANES, LANES)] - min_splat)

        def in_copy(j, p):
            return pltpu.make_async_copy(
                in_hbm.at[pl.ds(row0 + j * CHUNK, CHUNK)],
                in_bufs[p], sem_in[p])

        def out_copy(j, p):
            return pltpu.make_async_copy(
                in_bufs[p], out_hbm.at[pl.ds(row0 + j * CHUNK, CHUNK)],
                sem_out[p])

        def chunk_body(j, p, first):
            in_copy(j, p).wait()
            q = 1 - p

            @pl.when(j + 1 < NCHUNKS)
            def _():
                if not first:
                    out_copy(j, q).wait()
                compute_idx(j + 1, q)
                in_copy(j + 1, q).start()

            descs = [
                pltpu.async_copy(
                    table_spm.at[idx_bufs[p].at[pl.ds(g * GCHUNK, GCHUNK)]],
                    in_bufs[p].at[pl.ds(g * GCHUNK, GCHUNK)],
                    sem_g, add=True)
                for g in range(NGATHER)
            ]
            for d in descs:
                d.wait()
            out_copy(j, p).start()

        compute_idx(0, 0)
        in_copy(0, 0).start()
        chunk_body(0, 0, first=True)

        def pair(i, carry):
            chunk_body(2 * i - 1, 1, first=False)
            chunk_body(2 * i, 0, first=False)
            return carry

        lax.fori_loop(1, (NCHUNKS + 1) // 2, pair, 0)

        out_copy(NCHUNKS - 2, 1).wait()
        out_copy(NCHUNKS - 1, 0).wait()


@functools.partial(jax.jit, static_argnames=())
def kernel(input_encoded, timesteps, pos_encoding):
    in2d = input_encoded.reshape(N_ROWS, D)
    ts1d = timesteps.reshape(N_ROWS)

    mesh = plsc.VectorSubcoreMesh(core_axis_name="c", subcore_axis_name="s")
    run = pl.kernel(
        _sc_body,
        out_type=jax.ShapeDtypeStruct((N_ROWS, D), jnp.float32),
        mesh=mesh,
        scratch_types=[
            pltpu.VMEM((ROWS_PER_BATCH,), jnp.int32),
            [pltpu.VMEM((CHUNK,), jnp.int32) for _ in range(2)],
            [pltpu.VMEM((CHUNK, D), jnp.float32) for _ in range(2)],
            pltpu.VMEM_SHARED((L, D), jnp.float32),
            [pltpu.SemaphoreType.DMA for _ in range(2)],
            [pltpu.SemaphoreType.DMA for _ in range(2)],
            pltpu.SemaphoreType.DMA,
        ],
        compiler_params=pltpu.CompilerParams(use_tc_tiling_on_sc=False),
    )
    out2d = run(in2d, ts1d, pos_encoding)
    return out2d.reshape(B, S, T, D)
